# Initial kernel scaffold; baseline (speedup 1.0000x reference)
import jax
import jax.numpy as jnp
from jax.experimental import pallas as pl


def kernel(x, pos_table):
    raise NotImplementedError("write your pallas kernel here")


# speedup vs baseline: 1.6961x; 1.6961x over previous
import jax
import jax.numpy as jnp
from jax.experimental import pallas as pl

MAXLEN = 2048
EMBED_DIM = 2048
BATCH = 4
T_BLK = 256


def _add_body(x_ref, pos_ref, out_ref):
    out_ref[...] = x_ref[...] + pos_ref[...][None]


def kernel(x, pos_table):
    grid = (MAXLEN // T_BLK, BATCH)
    return pl.pallas_call(
        _add_body,
        grid=grid,
        in_specs=[
            pl.BlockSpec((1, T_BLK, EMBED_DIM), lambda t, b: (b, t, 0)),
            pl.BlockSpec((T_BLK, EMBED_DIM), lambda t, b: (t, 0)),
        ],
        out_specs=pl.BlockSpec((1, T_BLK, EMBED_DIM), lambda t, b: (b, t, 0)),
        out_shape=jax.ShapeDtypeStruct((BATCH, MAXLEN, EMBED_DIM), x.dtype),
    )(x, pos_table)
